# baseline (device time: 29359 ns/iter reference)
import jax
import jax.numpy as jnp
from jax import lax
from jax.experimental import pallas as pl
from jax.experimental.pallas import tpu as pltpu

N_DEV = 16
N_TOK = 1024
D_IN = 512
D_OUT = 1024
N_EXP = 64
E_LOCAL = 4
CAP = 12
SLOCAL = E_LOCAL * CAP
SPAD = 64
STOT = N_DEV * SPAD
SCPAD = 8


def _body(x_ref, route_ref, w_ref, out_ref, ag_ref, sc_ref, xv_ref, wv_ref,
          rv_ref, dssem, drsem, sssem, srsem, xsem, wsem, rsem2):
    f32 = jnp.float32
    bf16 = jnp.bfloat16
    i32 = jnp.int32
    my = lax.axis_index("i")

    cp_r = pltpu.make_async_copy(route_ref, rv_ref, rsem2)
    cp_x = pltpu.make_async_copy(x_ref, xv_ref, xsem)
    cp_w = pltpu.make_async_copy(w_ref, wv_ref, wsem)
    cp_r.start()
    cp_x.start()
    cp_w.start()

    barrier = pltpu.get_barrier_semaphore()
    for j in range(1, N_DEV):
        pl.semaphore_signal(barrier, inc=1,
                            device_id=(lax.rem(my + j, N_DEV),),
                            device_id_type=pl.DeviceIdType.MESH)

    cp_r.wait()
    route = rv_ref[...]
    e_iota = lax.broadcasted_iota(i32, (N_TOK, N_EXP), 1)
    sel = (route == e_iota).astype(bf16)
    r_iota = lax.broadcasted_iota(i32, (N_TOK, N_TOK), 0)
    c_iota = lax.broadcasted_iota(i32, (N_TOK, N_TOK), 1)
    tril = (r_iota >= c_iota).astype(bf16)
    occ64 = jnp.dot(tril, sel, preferred_element_type=f32)
    occ = jnp.sum(occ64 * sel.astype(f32), axis=1,
                  keepdims=True).astype(i32)
    kept = occ <= CAP

    rel_dev = ((route // E_LOCAL) - my) & (N_DEV - 1)
    rel_slot = rel_dev * SPAD + (route & (E_LOCAL - 1)) * CAP + occ - 1

    t_iota = lax.broadcasted_iota(i32, (N_TOK, SPAD), 1)
    pmy = ((rel_slot == t_iota) & kept).astype(bf16)
    cp_x.wait()
    xb = xv_ref[...].astype(bf16)
    xg = lax.dot_general(pmy, xb, (((0,), (0,)), ((), ())),
                         preferred_element_type=f32).astype(bf16)

    g_iota = lax.broadcasted_iota(i32, (SPAD, 1), 0)
    cp_w.wait()
    acc = jnp.zeros((SPAD, D_OUT), f32)
    for k in range(E_LOCAL):
        gmask = ((g_iota >= k * CAP) & (g_iota < (k + 1) * CAP))
        acc += jnp.dot(xg * gmask.astype(bf16), wv_ref[k].astype(bf16),
                       preferred_element_type=f32)

    amax = jnp.max(jnp.abs(acc))
    inv = jnp.where(amax > 0, 127.0 / amax, 0.0)
    q = jnp.clip(jnp.round(acc * inv), -127.0, 127.0)
    ag_ref[pl.ds(0, SPAD), :] = q.astype(jnp.int8)
    sc_ref[pl.ds(0, SCPAD), :] = jnp.full(
        (SCPAD, 1), jnp.where(amax > 0, amax / 127.0, 0.0), f32)

    pl.semaphore_wait(barrier, N_DEV - 1)

    sends = []
    for j in range(1, N_DEV):
        peer = (lax.rem(my + j, N_DEV),)
        m = N_DEV - j
        data = pltpu.make_async_remote_copy(
            src_ref=ag_ref.at[pl.ds(0, SPAD)],
            dst_ref=ag_ref.at[pl.ds(m * SPAD, SPAD)],
            send_sem=dssem.at[j - 1], recv_sem=drsem.at[j - 1],
            device_id=peer, device_id_type=pl.DeviceIdType.MESH,
        )
        scale = pltpu.make_async_remote_copy(
            src_ref=sc_ref.at[pl.ds(0, SCPAD)],
            dst_ref=sc_ref.at[pl.ds(m * SCPAD, SCPAD)],
            send_sem=sssem.at[j - 1], recv_sem=srsem.at[j - 1],
            device_id=peer, device_id_type=pl.DeviceIdType.MESH,
        )
        data.start()
        scale.start()
        sends += [data, scale]

    s_iota = lax.broadcasted_iota(i32, (N_TOK, STOT), 1)
    p = ((rel_slot == s_iota) & kept).astype(bf16)

    b_iota = lax.broadcasted_iota(i32, (STOT, N_DEV * SCPAD), 0)
    bm_iota = lax.broadcasted_iota(i32, (STOT, N_DEV * SCPAD), 1)
    rowsel = ((b_iota // SPAD) * SCPAD == bm_iota).astype(f32)

    for j in range(1, N_DEV):
        peer = (lax.rem(my - j + N_DEV, N_DEV),)
        m = N_DEV - j
        pltpu.make_async_remote_copy(
            src_ref=ag_ref.at[pl.ds(0, SPAD)],
            dst_ref=ag_ref.at[pl.ds(m * SPAD, SPAD)],
            send_sem=dssem.at[j - 1], recv_sem=drsem.at[j - 1],
            device_id=peer, device_id_type=pl.DeviceIdType.MESH,
        ).wait_recv()
        pltpu.make_async_remote_copy(
            src_ref=sc_ref.at[pl.ds(0, SCPAD)],
            dst_ref=sc_ref.at[pl.ds(m * SCPAD, SCPAD)],
            send_sem=sssem.at[j - 1], recv_sem=srsem.at[j - 1],
            device_id=peer, device_id_type=pl.DeviceIdType.MESH,
        ).wait_recv()

    rep = jnp.dot(rowsel, sc_ref[...], preferred_element_type=f32)
    agb = (ag_ref[...].astype(f32) * rep).astype(bf16)
    out_ref[...] = jnp.dot(p, agb, preferred_element_type=f32).astype(bf16)

    for rdma in sends:
        rdma.wait_send()


def kernel(x, router_W, route_idx, expert_W):
    del router_W
    return pl.pallas_call(
        _body,
        out_shape=jax.ShapeDtypeStruct((N_TOK, D_OUT), jnp.bfloat16),
        in_specs=[
            pl.BlockSpec(memory_space=pltpu.MemorySpace.HBM),
            pl.BlockSpec(memory_space=pltpu.MemorySpace.HBM),
            pl.BlockSpec(memory_space=pltpu.MemorySpace.HBM),
        ],
        out_specs=pl.BlockSpec(memory_space=pltpu.VMEM),
        scratch_shapes=[
            pltpu.VMEM((STOT, D_OUT), jnp.int8),
            pltpu.VMEM((N_DEV * SCPAD, 1), jnp.float32),
            pltpu.VMEM((N_TOK, D_IN), jnp.float32),
            pltpu.VMEM((E_LOCAL, D_IN, D_OUT), jnp.float32),
            pltpu.VMEM((N_TOK, 1), jnp.int32),
            pltpu.SemaphoreType.DMA((N_DEV - 1,)),
            pltpu.SemaphoreType.DMA((N_DEV - 1,)),
            pltpu.SemaphoreType.DMA((N_DEV - 1,)),
            pltpu.SemaphoreType.DMA((N_DEV - 1,)),
            pltpu.SemaphoreType.DMA,
            pltpu.SemaphoreType.DMA,
            pltpu.SemaphoreType.DMA,
        ],
        compiler_params=pltpu.CompilerParams(collective_id=0),
    )(x, route_idx, expert_W)


# device time: 28625 ns/iter; 1.0256x vs baseline; 1.0256x over previous
import jax
import jax.numpy as jnp
from jax import lax
from jax.experimental import pallas as pl
from jax.experimental.pallas import tpu as pltpu

N_DEV = 16
N_TOK = 1024
D_IN = 512
D_OUT = 1024
N_EXP = 64
E_LOCAL = 4
CAP = 12
SLOCAL = E_LOCAL * CAP
SPAD = 64
STOT = N_DEV * SPAD
SCPAD = 8
HALF = 8 * SPAD


def _body(x_ref, route_ref, w_ref, out_ref, ag_ref, sc_ref,
          dssem, drsem, sssem, srsem):
    f32 = jnp.float32
    bf16 = jnp.bfloat16
    i32 = jnp.int32
    my = lax.axis_index("i")

    barrier = pltpu.get_barrier_semaphore()
    for j in range(1, N_DEV):
        pl.semaphore_signal(barrier, inc=1,
                            device_id=(lax.rem(my + j, N_DEV),),
                            device_id_type=pl.DeviceIdType.MESH)

    route = route_ref[...]
    e_iota = lax.broadcasted_iota(i32, (N_TOK, N_EXP), 1)
    sel = (route == e_iota).astype(bf16)
    r_iota = lax.broadcasted_iota(i32, (N_TOK, N_TOK), 0)
    c_iota = lax.broadcasted_iota(i32, (N_TOK, N_TOK), 1)
    tril = (r_iota >= c_iota).astype(bf16)
    occ64 = jnp.dot(tril, sel, preferred_element_type=f32)
    occ = jnp.sum(occ64 * sel.astype(f32), axis=1,
                  keepdims=True).astype(i32)
    kept = occ <= CAP

    rel_dev = ((route // E_LOCAL) - my) & (N_DEV - 1)
    rel_slot = rel_dev * SPAD + (route & (E_LOCAL - 1)) * CAP + occ - 1

    t_iota = lax.broadcasted_iota(i32, (N_TOK, SPAD), 1)
    pmy = ((rel_slot == t_iota) & kept).astype(bf16)
    xb = x_ref[...].astype(bf16)
    xg = lax.dot_general(pmy, xb, (((0,), (0,)), ((), ())),
                         preferred_element_type=f32).astype(bf16)

    g_iota = lax.broadcasted_iota(i32, (SPAD, 1), 0)
    acc = jnp.zeros((SPAD, D_OUT), f32)
    for k in range(E_LOCAL):
        gmask = ((g_iota >= k * CAP) & (g_iota < (k + 1) * CAP))
        acc += jnp.dot(xg * gmask.astype(bf16), w_ref[k].astype(bf16),
                       preferred_element_type=f32)

    amax = jnp.max(jnp.abs(acc))
    inv = jnp.where(amax > 0, 127.0 / amax, 0.0)
    q = jnp.clip(jnp.round(acc * inv), -127.0, 127.0)
    ag_ref[pl.ds(0, SPAD), :] = q.astype(jnp.int8)
    sc_ref[pl.ds(0, SCPAD), :] = jnp.full(
        (SCPAD, 1), jnp.where(amax > 0, amax / 127.0, 0.0), f32)

    pl.semaphore_wait(barrier, N_DEV - 1)

    sends = []
    for j in range(1, N_DEV):
        peer = (lax.rem(my + j, N_DEV),)
        m = N_DEV - j
        data = pltpu.make_async_remote_copy(
            src_ref=ag_ref.at[pl.ds(0, SPAD)],
            dst_ref=ag_ref.at[pl.ds(m * SPAD, SPAD)],
            send_sem=dssem.at[j - 1], recv_sem=drsem.at[j - 1],
            device_id=peer, device_id_type=pl.DeviceIdType.MESH,
        )
        scale = pltpu.make_async_remote_copy(
            src_ref=sc_ref.at[pl.ds(0, SCPAD)],
            dst_ref=sc_ref.at[pl.ds(m * SCPAD, SCPAD)],
            send_sem=sssem.at[j - 1], recv_sem=srsem.at[j - 1],
            device_id=peer, device_id_type=pl.DeviceIdType.MESH,
        )
        data.start()
        scale.start()
        sends += [data, scale]

    s_iota = lax.broadcasted_iota(i32, (N_TOK, STOT), 1)
    p = ((rel_slot == s_iota) & kept).astype(bf16)

    b_iota = lax.broadcasted_iota(i32, (STOT, N_DEV * SCPAD), 0)
    bm_iota = lax.broadcasted_iota(i32, (STOT, N_DEV * SCPAD), 1)
    rowsel = ((b_iota // SPAD) * SCPAD == bm_iota).astype(f32)

    def wait_group(js):
        for j in js:
            peer = (lax.rem(my - j + N_DEV, N_DEV),)
            m = N_DEV - j
            pltpu.make_async_remote_copy(
                src_ref=ag_ref.at[pl.ds(0, SPAD)],
                dst_ref=ag_ref.at[pl.ds(m * SPAD, SPAD)],
                send_sem=dssem.at[j - 1], recv_sem=drsem.at[j - 1],
                device_id=peer, device_id_type=pl.DeviceIdType.MESH,
            ).wait_recv()
            pltpu.make_async_remote_copy(
                src_ref=sc_ref.at[pl.ds(0, SCPAD)],
                dst_ref=sc_ref.at[pl.ds(m * SCPAD, SCPAD)],
                send_sem=sssem.at[j - 1], recv_sem=srsem.at[j - 1],
                device_id=peer, device_id_type=pl.DeviceIdType.MESH,
            ).wait_recv()

    def half_out(lo):
        rep = jnp.dot(rowsel[lo:lo + HALF], sc_ref[...],
                      preferred_element_type=f32)
        agb = (ag_ref[pl.ds(lo, HALF), :].astype(f32) * rep).astype(bf16)
        return jnp.dot(p[:, lo:lo + HALF], agb,
                       preferred_element_type=f32)

    wait_group(range(9, N_DEV))
    part1 = half_out(0)
    wait_group(range(1, 9))
    out_ref[...] = (part1 + half_out(HALF)).astype(bf16)

    for rdma in sends:
        rdma.wait_send()


def kernel(x, router_W, route_idx, expert_W):
    del router_W
    return pl.pallas_call(
        _body,
        out_shape=jax.ShapeDtypeStruct((N_TOK, D_OUT), jnp.bfloat16),
        in_specs=[pl.BlockSpec(memory_space=pltpu.VMEM)] * 3,
        out_specs=pl.BlockSpec(memory_space=pltpu.VMEM),
        scratch_shapes=[
            pltpu.VMEM((STOT, D_OUT), jnp.int8),
            pltpu.VMEM((N_DEV * SCPAD, 1), jnp.float32),
            pltpu.SemaphoreType.DMA((N_DEV - 1,)),
            pltpu.SemaphoreType.DMA((N_DEV - 1,)),
            pltpu.SemaphoreType.DMA((N_DEV - 1,)),
            pltpu.SemaphoreType.DMA((N_DEV - 1,)),
        ],
        compiler_params=pltpu.CompilerParams(collective_id=0),
    )(x, route_idx, expert_W)
